# Initial kernel scaffold; baseline (speedup 1.0000x reference)
import functools

import jax
import jax.numpy as jnp
from jax import lax
from jax.experimental import pallas as pl
from jax.experimental.pallas import tpu as pltpu
from jax.experimental.pallas import tpu_sc as plsc

N = 10000
D = 128
G = 512
E = 320000
NC = 2
NS = 16
NW = NC * NS
C = 125
K = E // (NW * C)
RPS = N // NS
ZR = 125
DEG_PAD = 10240
DPS = DEG_PAD // NS
PC = 80
PK = N // PC
GPS = G // NS

_mesh = plsc.VectorSubcoreMesh(core_axis_name="c", subcore_axis_name="s")


def _zero_rows(buf, nrows):
    zero = jnp.zeros((16,), jnp.float32)

    def body(t, carry):
        i = t // (D // 16)
        j = t % (D // 16)
        buf[i, pl.ds(j * 16, 16)] = zero
        return carry

    lax.fori_loop(0, nrows * (D // 16), body, 0)


@functools.partial(
    pl.kernel,
    mesh=_mesh,
    out_type=jax.ShapeDtypeStruct((NC, DEG_PAD), jnp.float32),
    scratch_types=[
        pltpu.VMEM_SHARED((DEG_PAD,), jnp.float32),
        pltpu.VMEM((K, C), jnp.int32),
        pltpu.VMEM((128,), jnp.float32),
        pltpu.VMEM((DPS,), jnp.float32),
    ],
)
def _deg_kernel(dst_hbm, degp_hbm, dacc, idx_d, ones_v, zb):
    c = lax.axis_index("c")
    s = lax.axis_index("s")
    w = s * NC + c
    one = jnp.ones((16,), jnp.float32)
    zero = jnp.zeros((16,), jnp.float32)
    for t in range(128 // 16):
        ones_v[pl.ds(t * 16, 16)] = one

    def zbody(t, carry):
        zb[pl.ds(t * 16, 16)] = zero
        return carry

    lax.fori_loop(0, DPS // 16, zbody, 0)
    pltpu.sync_copy(zb, dacc.at[pl.ds(s * DPS, DPS)])
    plsc.subcore_barrier()
    pltpu.sync_copy(dst_hbm.at[w], idx_d)

    def step(j, carry):
        pltpu.sync_copy(ones_v.at[pl.ds(0, C)], dacc.at[idx_d.at[j]], add=True)
        return carry

    lax.fori_loop(0, K, step, 0)
    plsc.subcore_barrier()
    pltpu.sync_copy(dacc.at[pl.ds(s * DPS, DPS)],
                    degp_hbm.at[c].at[pl.ds(s * DPS, DPS)])


@functools.partial(
    pl.kernel,
    mesh=_mesh,
    out_type=jax.ShapeDtypeStruct((NC, N, D), jnp.float32),
    scratch_types=[
        pltpu.VMEM_SHARED((N, D), jnp.float32),
        pltpu.VMEM((K, C), jnp.int32),
        pltpu.VMEM((K, C), jnp.int32),
        pltpu.VMEM((2, C, D), jnp.float32),
        pltpu.VMEM((ZR, D), jnp.float32),
        pltpu.SemaphoreType.DMA,
        pltpu.SemaphoreType.DMA,
    ],
)
def _edge_kernel(y_hbm, src_hbm, dst_hbm, z_hbm, acc, idx_s, idx_d, rows,
                 zbuf, sem0, sem1):
    c = lax.axis_index("c")
    s = lax.axis_index("s")
    w = s * NC + c
    _zero_rows(zbuf, ZR)
    for r in range(RPS // ZR):
        pltpu.sync_copy(zbuf, acc.at[pl.ds(s * RPS + r * ZR, ZR)])
    plsc.subcore_barrier()
    pltpu.sync_copy(src_hbm.at[w], idx_s)
    pltpu.sync_copy(dst_hbm.at[w], idx_d)

    pltpu.async_copy(y_hbm.at[idx_s.at[0]], rows.at[0], sem0)

    def pair(p, carry):
        j0 = p * 2
        pltpu.async_copy(y_hbm.at[idx_s.at[j0 + 1]], rows.at[1], sem1)
        pltpu.make_async_copy(y_hbm.at[idx_s.at[j0]], rows.at[0], sem0).wait()
        pltpu.sync_copy(rows.at[0], acc.at[idx_d.at[j0]], add=True)

        @pl.when(j0 + 2 < K)
        def _():
            pltpu.async_copy(y_hbm.at[idx_s.at[j0 + 2]], rows.at[0], sem0)

        pltpu.make_async_copy(y_hbm.at[idx_s.at[j0 + 1]], rows.at[1],
                              sem1).wait()
        pltpu.sync_copy(rows.at[1], acc.at[idx_d.at[j0 + 1]], add=True)
        return carry

    lax.fori_loop(0, K // 2, pair, 0)
    plsc.subcore_barrier()
    pltpu.sync_copy(acc.at[pl.ds(s * RPS, RPS)],
                    z_hbm.at[c].at[pl.ds(s * RPS, RPS)])


@functools.partial(
    pl.kernel,
    mesh=_mesh,
    out_type=(
        jax.ShapeDtypeStruct((NC, G, D), jnp.float32),
        jax.ShapeDtypeStruct((NC, G, D), jnp.float32),
        jax.ShapeDtypeStruct((NC, G, D), jnp.float32),
    ),
    scratch_types=[
        pltpu.VMEM_SHARED((G, D), jnp.float32),
        pltpu.VMEM_SHARED((G, D), jnp.float32),
        pltpu.VMEM_SHARED((G, D), jnp.float32),
        pltpu.VMEM((4, PC), jnp.int32),
        pltpu.VMEM((PC, D), jnp.float32),
        pltpu.VMEM((PC, D), jnp.float32),
        pltpu.VMEM((PC, D), jnp.float32),
    ],
)
def _pool_kernel(x_hbm, h1_hbm, h2_hbm, batch_hbm, px_hbm, p1_hbm, p2_hbm,
                 ax, a1, a2, bidx, rx, r1, r2):
    c = lax.axis_index("c")
    s = lax.axis_index("s")
    w = s * NC + c
    _zero_rows(rx, PC)
    for a in (ax, a1, a2):
        pltpu.sync_copy(rx.at[pl.ds(0, GPS)], a.at[pl.ds(s * GPS, GPS)])
    plsc.subcore_barrier()
    for t in range(4):
        j = w + NW * t

        @pl.when(j < PK)
        def _():
            pltpu.sync_copy(batch_hbm.at[j], bidx.at[t])
            base = j * PC
            pltpu.sync_copy(x_hbm.at[pl.ds(base, PC)], rx)
            pltpu.sync_copy(rx, ax.at[bidx.at[t]], add=True)
            pltpu.sync_copy(h1_hbm.at[pl.ds(base, PC)], r1)
            pltpu.sync_copy(r1, a1.at[bidx.at[t]], add=True)
            pltpu.sync_copy(h2_hbm.at[pl.ds(base, PC)], r2)
            pltpu.sync_copy(r2, a2.at[bidx.at[t]], add=True)

    plsc.subcore_barrier()
    for a, out in ((ax, px_hbm), (a1, p1_hbm), (a2, p2_hbm)):
        pltpu.sync_copy(a.at[pl.ds(s * GPS, GPS)],
                        out.at[c].at[pl.ds(s * GPS, GPS)])


def _tc1_body(degp2_ref, x_ref, w1_ref, dinv_ref, y1_ref):
    deg = degp2_ref[:, 0:1] + degp2_ref[:, 1:2] + 1.0
    dinv = lax.rsqrt(deg)
    dinv_ref[...] = dinv
    xw = jnp.dot(x_ref[...], w1_ref[...], preferred_element_type=jnp.float32)
    y1_ref[...] = xw * dinv


def _tc2_body(z_ref, y1_ref, dinv_ref, b1_ref, w2_ref, h1_ref, y2_ref):
    dinv = dinv_ref[...]
    zt = z_ref[0] + z_ref[1] + y1_ref[...]
    h1 = jnp.tanh(zt * dinv + b1_ref[...])
    h1_ref[...] = h1
    y2_ref[...] = jnp.dot(h1, w2_ref[...],
                          preferred_element_type=jnp.float32) * dinv


def _tc3_body(z_ref, y2_ref, dinv_ref, b2_ref, h2_ref):
    zt = z_ref[0] + z_ref[1] + y2_ref[...]
    h2_ref[...] = jnp.tanh(zt * dinv_ref[...] + b2_ref[...])


def _tc4_body(px_ref, p1_ref, p2_ref, wfc_ref, bfc_ref, out_ref):
    pooled = jnp.concatenate(
        [px_ref[0] + px_ref[1], p1_ref[0] + p1_ref[1], p2_ref[0] + p2_ref[1]],
        axis=1)
    out_ref[...] = jnp.dot(pooled, wfc_ref[...],
                           preferred_element_type=jnp.float32) + bfc_ref[...]


def kernel(x, edge_index, batch, edge_index_cg, W1, b1, W2, b2, Wfc, bfc):
    src = edge_index[0].reshape(NW, K, C)
    dst = edge_index[1].reshape(NW, K, C)
    batch2 = batch.reshape(PK, PC)

    degp = _deg_kernel(dst)
    degp2 = degp[:, :N].T

    dinv, y1 = pl.pallas_call(
        _tc1_body,
        out_shape=(jax.ShapeDtypeStruct((N, 1), jnp.float32),
                   jax.ShapeDtypeStruct((N, D), jnp.float32)),
    )(degp2, x, W1)

    z1 = _edge_kernel(y1, src, dst)

    h1, y2 = pl.pallas_call(
        _tc2_body,
        out_shape=(jax.ShapeDtypeStruct((N, D), jnp.float32),
                   jax.ShapeDtypeStruct((N, D), jnp.float32)),
    )(z1, y1, dinv, b1, W2)

    z2 = _edge_kernel(y2, src, dst)

    h2 = pl.pallas_call(
        _tc3_body,
        out_shape=jax.ShapeDtypeStruct((N, D), jnp.float32),
    )(z2, y2, dinv, b2)

    px, p1, p2 = _pool_kernel(x, h1, h2, batch2)

    out = pl.pallas_call(
        _tc4_body,
        out_shape=jax.ShapeDtypeStruct((G, D * 3), jnp.float32),
    )(px, p1, p2, Wfc, bfc)
    return out


# speedup vs baseline: 41.2340x; 41.2340x over previous
import functools

import jax
import jax.numpy as jnp
from jax import lax
from jax.experimental import pallas as pl
from jax.experimental.pallas import tpu as pltpu
from jax.experimental.pallas import tpu_sc as plsc

N = 10000
D = 128
G = 512
E = 320000
NC = 2
NS = 16
NW = NC * NS
C = 80
K = E // (NW * C)
EPW = E // NW
CB = 5
NP = 10240
RPS = NP // NS
ZR = 80
DEG_PAD = 10240
DPS = DEG_PAD // NS


def _deg_body(e_hbm, degp_hbm, dacc, idx_d, ones_v, zb, dsem):
    c = lax.axis_index("c")
    s = lax.axis_index("s")
    w = s * NC + c
    one = jnp.ones((16,), jnp.float32)
    zero = jnp.zeros((16,), jnp.float32)
    for t in range(128 // 16):
        ones_v[pl.ds(t * 16, 16)] = one

    def zbody(t, carry):
        zb[pl.ds(t * 16, 16)] = zero
        return carry

    lax.fori_loop(0, DPS // 16, zbody, 0)
    pltpu.sync_copy(zb, dacc.at[pl.ds(s * DPS, DPS)])
    plsc.subcore_barrier()
    pltpu.sync_copy(e_hbm.at[1].at[pl.ds(w * EPW, EPW)], idx_d)

    def step(j, carry):
        pltpu.async_copy(ones_v.at[pl.ds(0, C)],
                         dacc.at[idx_d.at[pl.ds(j * C, C)]], dsem, add=True)
        return carry

    lax.fori_loop(0, K, step, 0)

    def drain(j, carry):
        pltpu.make_async_copy(ones_v.at[pl.ds(0, C)],
                              dacc.at[idx_d.at[pl.ds(j * C, C)]], dsem).wait()
        return carry

    lax.fori_loop(0, K, drain, 0)
    plsc.subcore_barrier()
    pltpu.sync_copy(dacc.at[pl.ds(s * DPS, DPS)],
                    degp_hbm.at[c].at[pl.ds(s * DPS, DPS)])


def _edge_body(y_hbm, e_hbm, z_hbm, acc, idx_s, idx_d, rows,
               sem0, sem1, sem2, sem3, sem4):
    c = lax.axis_index("c")
    s = lax.axis_index("s")
    w = s * NC + c
    sems = (sem0, sem1, sem2, sem3, sem4)

    zero = jnp.zeros((32,), jnp.bfloat16)

    def zbody(t, carry):
        i = t // (D // 32)
        q = t % (D // 32)
        rows[0, i, pl.ds(q * 32, 32)] = zero
        return carry

    lax.fori_loop(0, ZR * (D // 32), zbody, 0)
    for r in range(RPS // ZR):
        pltpu.sync_copy(rows.at[0].at[pl.ds(0, ZR)],
                        acc.at[pl.ds(s * RPS + r * ZR, ZR)])
    plsc.subcore_barrier()

    pltpu.sync_copy(e_hbm.at[0].at[pl.ds(w * EPW, EPW)], idx_s)
    pltpu.sync_copy(e_hbm.at[1].at[pl.ds(w * EPW, EPW)], idx_d)

    for b in range(CB - 1):
        pltpu.async_copy(y_hbm.at[idx_s.at[pl.ds(b * C, C)]], rows.at[b],
                         sems[b])

    def quint(p, carry):
        j0 = p * CB
        for i in range(CB):
            j = j0 + i
            pltpu.make_async_copy(y_hbm.at[idx_s.at[pl.ds(j * C, C)]],
                                  rows.at[i], sems[i]).wait()
            nb = (i + CB - 1) % CB

            @pl.when(j + CB - 1 < K)
            def _():
                pltpu.async_copy(
                    y_hbm.at[idx_s.at[pl.ds((j + CB - 1) * C, C)]],
                    rows.at[nb], sems[nb])

            pltpu.sync_copy(rows.at[i], acc.at[idx_d.at[pl.ds(j * C, C)]],
                            add=True)
        return carry

    lax.fori_loop(0, K // CB, quint, 0)

    plsc.subcore_barrier()
    pltpu.sync_copy(acc.at[pl.ds(s * RPS, RPS)],
                    z_hbm.at[c].at[pl.ds(s * RPS, RPS)])


@functools.cache
def _sc_kernels():
    mesh = plsc.VectorSubcoreMesh(core_axis_name="c", subcore_axis_name="s",
                                  num_cores=NC, num_subcores=NS)
    deg = pl.kernel(
        _deg_body,
        mesh=mesh,
        out_type=jax.ShapeDtypeStruct((NC, DEG_PAD), jnp.float32),
        compiler_params=pltpu.CompilerParams(use_tc_tiling_on_sc=False),
        scratch_types=[
            pltpu.VMEM_SHARED((DEG_PAD,), jnp.float32),
            pltpu.VMEM((EPW,), jnp.int32),
            pltpu.VMEM((128,), jnp.float32),
            pltpu.VMEM((DPS,), jnp.float32),
            pltpu.SemaphoreType.DMA,
        ],
    )
    edge = pl.kernel(
        _edge_body,
        mesh=mesh,
        out_type=jax.ShapeDtypeStruct((NC, NP, D), jnp.bfloat16),
        compiler_params=pltpu.CompilerParams(use_tc_tiling_on_sc=False),
        scratch_types=[
            pltpu.VMEM_SHARED((NP, D), jnp.bfloat16),
            pltpu.VMEM((EPW,), jnp.int32),
            pltpu.VMEM((EPW,), jnp.int32),
            pltpu.VMEM((CB, C, D), jnp.bfloat16),
            pltpu.SemaphoreType.DMA,
            pltpu.SemaphoreType.DMA,
            pltpu.SemaphoreType.DMA,
            pltpu.SemaphoreType.DMA,
            pltpu.SemaphoreType.DMA,
        ],
    )
    return deg, edge

def _tc1_body(degp_ref, x_ref, w1_ref, dinv_ref, y1_ref):
    dsum = degp_ref[0:1, :N] + degp_ref[1:2, :N] + 1.0
    dinv = jnp.transpose(lax.rsqrt(dsum))
    dinv_ref[...] = dinv
    xw = jnp.dot(x_ref[...], w1_ref[...], preferred_element_type=jnp.float32)
    y1_ref[...] = (xw * dinv).astype(jnp.bfloat16)


def _tc2_body(z_ref, y1_ref, dinv_ref, b1_ref, w2_ref, h1_ref, y2_ref):
    dinv = dinv_ref[...]
    zt = (z_ref[0, :N].astype(jnp.float32) + z_ref[1, :N].astype(jnp.float32)
          + y1_ref[...].astype(jnp.float32))
    h1 = jnp.tanh(zt * dinv + b1_ref[...])
    h1_ref[...] = h1.astype(jnp.bfloat16)
    y2_ref[...] = (jnp.dot(h1, w2_ref[...],
                           preferred_element_type=jnp.float32) *
                   dinv).astype(jnp.bfloat16)


def _tc34_body(z_ref, y2_ref, dinv_ref, b2_ref, batch_ref, x_ref, h1_ref,
               wfc_ref, bfc_ref, out_ref):
    zt = (z_ref[0, :N].astype(jnp.float32) + z_ref[1, :N].astype(jnp.float32)
          + y2_ref[...].astype(jnp.float32))
    h2 = jnp.tanh(zt * dinv_ref[...] + b2_ref[...])
    gids = jax.lax.broadcasted_iota(jnp.int32, (G, N), 0)
    st = (batch_ref[...] == gids).astype(jnp.bfloat16)
    dn = (((1,), (0,)), ((), ()))
    px = jax.lax.dot_general(st, x_ref[...].astype(jnp.bfloat16), dn,
                             preferred_element_type=jnp.float32)
    p1 = jax.lax.dot_general(st, h1_ref[...], dn,
                             preferred_element_type=jnp.float32)
    p2 = jax.lax.dot_general(st, h2.astype(jnp.bfloat16), dn,
                             preferred_element_type=jnp.float32)
    pooled = jnp.concatenate([px, p1, p2], axis=1)
    out_ref[...] = jnp.dot(pooled, wfc_ref[...],
                           preferred_element_type=jnp.float32) + bfc_ref[...]


def kernel(x, edge_index, batch, edge_index_cg, W1, b1, W2, b2, Wfc, bfc):
    _deg_kernel, _edge_kernel = _sc_kernels()
    degp = _deg_kernel(edge_index)

    dinv, y1 = pl.pallas_call(
        _tc1_body,
        out_shape=(jax.ShapeDtypeStruct((N, 1), jnp.float32),
                   jax.ShapeDtypeStruct((N, D), jnp.bfloat16)),
    )(degp, x, W1)

    z1 = _edge_kernel(y1, edge_index)

    h1, y2 = pl.pallas_call(
        _tc2_body,
        out_shape=(jax.ShapeDtypeStruct((N, D), jnp.bfloat16),
                   jax.ShapeDtypeStruct((N, D), jnp.bfloat16)),
    )(z1, y1, dinv, b1, W2)

    z2 = _edge_kernel(y2, edge_index)

    out = pl.pallas_call(
        _tc34_body,
        out_shape=jax.ShapeDtypeStruct((G, D * 3), jnp.float32),
    )(z2, y2, dinv, b2, batch.reshape(1, N), x, h1, Wfc, bfc)
    return out
